# baseline (speedup 1.0000x reference)
import functools

import jax
import jax.numpy as jnp
from jax import lax
from jax.experimental import pallas as pl
from jax.experimental.pallas import tpu as pltpu
from jax.experimental.pallas import tpu_sc as plsc


@functools.lru_cache(maxsize=None)
def _make_rel_pos_kernel(q_len, v_len, in_dim, out_dim):
    info = plsc.get_sparse_core_info()
    nc, ns = info.num_cores, info.num_subcores
    nw = nc * ns

    mp = (in_dim - 1) // 2
    pre = q_len - 1 - mp
    suf_start = pre + in_dim
    r_len = q_len + v_len - 1
    assert suf_start + (v_len - 1 - mp) == r_len
    assert q_len % nw == 0 and nw % 8 == 0 and in_dim % 8 == 1
    assert out_dim % 8 == 0 and v_len % 128 == 0
    rows_per_w = q_len // nw
    n_tk = out_dim // 8
    n_tj = v_len // 128

    wd = r_len - 7
    assert wd % 8 == 0 and wd <= 131071 // out_dim
    ph_w = in_dim + 7
    pre_end = pre + 1
    suf_fill = ((suf_start - 7) // 8) * 8
    assert pre_end % 8 == 0 and pre_end >= 16 and suf_fill + 8 >= suf_start - 7
    assert wd % 8 == 0 and (wd - suf_fill) % 8 == 0

    mesh = plsc.VectorSubcoreMesh(core_axis_name="c", subcore_axis_name="s")

    unroll = 2
    assert rows_per_w % unroll == 0

    @functools.partial(
        pl.kernel,
        out_type=jax.ShapeDtypeStruct((q_len, n_tk, n_tj, 8, 128),
                                      jnp.float32),
        mesh=mesh,
        scratch_types=[
            pltpu.VMEM((out_dim, wd), jnp.float32),
            pltpu.SemaphoreType.DMA,
        ],
        compiler_params=pltpu.CompilerParams(use_tc_tiling_on_sc=False),
    )
    def rel_pos(phases_hbm, runs_hbm, out_hbm, w_v, sem):
        wid = lax.axis_index("s") * nc + lax.axis_index("c")
        off = lax.rem(jnp.int32(q_len - 1) - wid, jnp.int32(8))
        u_b = jnp.int32(pre) - off
        delta = lax.rem(u_b, jnp.int32(8))
        a = pl.multiple_of(u_b - delta, 8)

        pltpu.sync_copy(runs_hbm.at[:, pl.ds(0, pre_end)],
                        w_v.at[:, pl.ds(0, pre_end)])
        pltpu.sync_copy(runs_hbm.at[:, pl.ds(pre_end, wd - suf_fill)],
                        w_v.at[:, pl.ds(suf_fill, wd - suf_fill)])
        pltpu.sync_copy(phases_hbm.at[:, delta], w_v.at[:, pl.ds(a, ph_w)])

        def fire(t, carry):
            handles = []
            for r in range(unroll):
                i = wid + (t * unroll + r) * nw
                u0 = pl.multiple_of(jnp.int32(q_len - 1) - i - off, 8)
                for tk in range(n_tk):
                    for tj in range(n_tj):
                        src = w_v.at[
                            pl.ds(8 * tk, 8),
                            pl.ds(pl.multiple_of(u0 + 128 * tj, 8), 128)]
                        handles.append(
                            pltpu.async_copy(src, out_hbm.at[i, tk, tj], sem))
            for h in handles:
                h.wait()
            return carry

        lax.fori_loop(0, rows_per_w // unroll, fire, 0)

    return rel_pos


def kernel(q, v, embeddings):
    q_len = q.shape[1]
    v_len = v.shape[1]
    in_dim, out_dim = embeddings.shape
    rel_pos = _make_rel_pos_kernel(q_len, v_len, in_dim, out_dim)

    emb_t = embeddings.T
    first = emb_t[:, :1]
    last = emb_t[:, -1:]
    ph_w = in_dim + 7
    base = jnp.concatenate(
        [jnp.broadcast_to(first, (out_dim, 7)), emb_t,
         jnp.broadcast_to(last, (out_dim, 7))], axis=1)
    idx = (7 - jnp.arange(8))[:, None] + jnp.arange(ph_w)[None, :]
    phases = base[:, idx]
    mp = (in_dim - 1) // 2
    pre_end = q_len - mp
    r_len = q_len + v_len - 1
    wd = r_len - 7
    suf_fill = ((pre_end - 1 + in_dim - 7) // 8) * 8
    runs = jnp.concatenate(
        [jnp.broadcast_to(first, (out_dim, pre_end)),
         jnp.broadcast_to(last, (out_dim, wd - suf_fill))], axis=1)

    f5 = rel_pos(phases, runs)
    out_t = f5.transpose(0, 1, 3, 2, 4).reshape(q_len, out_dim, v_len)
    return out_t.transpose(0, 2, 1)


# speedup vs baseline: 1.0707x; 1.0707x over previous
import functools

import jax
import jax.numpy as jnp
from jax import lax
from jax.experimental import pallas as pl
from jax.experimental.pallas import tpu as pltpu
from jax.experimental.pallas import tpu_sc as plsc


@functools.lru_cache(maxsize=None)
def _make_rel_pos_kernel(q_len, v_len, in_dim, out_dim):
    info = plsc.get_sparse_core_info()
    nc, ns = info.num_cores, info.num_subcores
    nw = nc * ns

    mp = (in_dim - 1) // 2
    pre = q_len - 1 - mp
    suf_start = pre + in_dim
    r_len = q_len + v_len - 1
    assert suf_start + (v_len - 1 - mp) == r_len
    assert q_len % nw == 0 and nw % 8 == 0 and in_dim % 8 == 1
    assert out_dim % 8 == 0 and v_len % 128 == 0
    rows_per_w = q_len // nw
    n_tk = out_dim // 8
    n_tj = v_len // 128

    wd = r_len - 7
    assert wd % 8 == 0 and wd <= 131071 // out_dim
    ph_w = in_dim + 7
    pre_end = pre + 1
    suf_fill = ((suf_start - 7) // 8) * 8
    assert pre_end % 8 == 0 and pre_end >= 16 and suf_fill + 8 >= suf_start - 7
    assert wd % 8 == 0 and (wd - suf_fill) % 8 == 0

    mesh = plsc.VectorSubcoreMesh(core_axis_name="c", subcore_axis_name="s")

    unroll = 1
    assert rows_per_w % unroll == 0

    @functools.partial(
        pl.kernel,
        out_type=jax.ShapeDtypeStruct((q_len, n_tk, n_tj, 8, 128),
                                      jnp.float32),
        mesh=mesh,
        scratch_types=[
            pltpu.VMEM((out_dim, wd), jnp.float32),
            pltpu.SemaphoreType.DMA,
        ],
        compiler_params=pltpu.CompilerParams(use_tc_tiling_on_sc=False),
    )
    def rel_pos(phases_hbm, runs_hbm, out_hbm, w_v, sem):
        wid = lax.axis_index("s") * nc + lax.axis_index("c")
        off = lax.rem(jnp.int32(q_len - 1) - wid, jnp.int32(8))
        u_b = jnp.int32(pre) - off
        delta = lax.rem(u_b, jnp.int32(8))
        a = pl.multiple_of(u_b - delta, 8)

        pltpu.sync_copy(runs_hbm.at[:, pl.ds(0, pre_end)],
                        w_v.at[:, pl.ds(0, pre_end)])
        pltpu.sync_copy(runs_hbm.at[:, pl.ds(pre_end, wd - suf_fill)],
                        w_v.at[:, pl.ds(suf_fill, wd - suf_fill)])
        pltpu.sync_copy(phases_hbm.at[:, delta], w_v.at[:, pl.ds(a, ph_w)])

        def fire(t, carry):
            handles = []
            for r in range(unroll):
                i = wid + (t * unroll + r) * nw
                u0 = pl.multiple_of(jnp.int32(q_len - 1) - i - off, 8)
                for tk in range(n_tk):
                    for tj in range(n_tj):
                        src = w_v.at[
                            pl.ds(8 * tk, 8),
                            pl.ds(pl.multiple_of(u0 + 128 * tj, 8), 128)]
                        handles.append(
                            pltpu.async_copy(src, out_hbm.at[i, tk, tj], sem))
            for h in handles:
                h.wait()
            return carry

        lax.fori_loop(0, rows_per_w // unroll, fire, 0)

    return rel_pos


def kernel(q, v, embeddings):
    q_len = q.shape[1]
    v_len = v.shape[1]
    in_dim, out_dim = embeddings.shape
    rel_pos = _make_rel_pos_kernel(q_len, v_len, in_dim, out_dim)

    emb_t = embeddings.T
    first = emb_t[:, :1]
    last = emb_t[:, -1:]
    ph_w = in_dim + 7
    base = jnp.concatenate(
        [jnp.broadcast_to(first, (out_dim, 7)), emb_t,
         jnp.broadcast_to(last, (out_dim, 7))], axis=1)
    idx = (7 - jnp.arange(8))[:, None] + jnp.arange(ph_w)[None, :]
    phases = base[:, idx]
    mp = (in_dim - 1) // 2
    pre_end = q_len - mp
    r_len = q_len + v_len - 1
    wd = r_len - 7
    suf_fill = ((pre_end - 1 + in_dim - 7) // 8) * 8
    runs = jnp.concatenate(
        [jnp.broadcast_to(first, (out_dim, pre_end)),
         jnp.broadcast_to(last, (out_dim, wd - suf_fill))], axis=1)

    f5 = rel_pos(phases, runs)
    out_t = f5.transpose(0, 1, 3, 2, 4).reshape(q_len, out_dim, v_len)
    return out_t.transpose(0, 2, 1)
